# Initial kernel scaffold; baseline (speedup 1.0000x reference)
import jax
import jax.numpy as jnp
from jax.experimental import pallas as pl


def kernel(x, edge_index, W1, b1, W2, b2, W3, b3):
    raise NotImplementedError("write your pallas kernel here")


# speedup vs baseline: 13.4605x; 13.4605x over previous
import functools

import jax
import jax.numpy as jnp
from jax import lax
from jax.experimental import pallas as pl
from jax.experimental.pallas import tpu as pltpu
from jax.experimental.pallas import tpu_sc as plsc

N = 10000
E = 320000
D_IN = 128
H = 16
C = 64

NC = 2
NS = 16
NW = NC * NS
EPT = E // NW
BLK = 80
NB = EPT // BLK
NP = 10240
RPT = NP // NS

_mesh = lambda: plsc.VectorSubcoreMesh(core_axis_name="c", subcore_axis_name="s")


def _make_prop(W):
    @functools.partial(
        pl.kernel,
        out_type=jax.ShapeDtypeStruct((NC, NP, W), jnp.float32),
        mesh=_mesh(),
        compiler_params=pltpu.CompilerParams(use_tc_tiling_on_sc=False),
        scratch_types=[
            pltpu.VMEM((BLK,), jnp.int32),
            pltpu.VMEM((BLK,), jnp.int32),
            pltpu.VMEM((BLK, W), jnp.float32),
            pltpu.VMEM((RPT, W), jnp.float32),
            pltpu.VMEM_SHARED((NP, W), jnp.float32),
            pltpu.SemaphoreType.DMA,
        ],
    )
    def prop(g_hbm, src_hbm, dst_hbm, out_hbm, src_v, dst_v, rows_v, zb_v,
             acc_sh, sem):
        cid = lax.axis_index("c")
        sid = lax.axis_index("s")
        wid = cid * NS + sid

        zvec = jnp.zeros((16,), jnp.float32)

        def zrow(r, carry):
            for c in range(W // 16):
                zb_v[r, pl.ds(c * 16, 16)] = zvec
            return carry

        lax.fori_loop(0, RPT, zrow, 0)
        pltpu.sync_copy(zb_v, acc_sh.at[pl.ds(sid * RPT, RPT)])
        plsc.subcore_barrier()

        ebase = wid * EPT

        def step(j, carry):
            off = ebase + j * BLK
            pltpu.sync_copy(src_hbm.at[pl.ds(off, BLK)], src_v)
            pltpu.sync_copy(dst_hbm.at[pl.ds(off, BLK)], dst_v)
            pltpu.async_copy(g_hbm.at[src_v], rows_v, sem).wait()
            pltpu.sync_copy(rows_v, acc_sh.at[dst_v], add=True)
            return carry

        lax.fori_loop(0, NB, step, 0)

        plsc.subcore_barrier()
        pltpu.sync_copy(acc_sh.at[pl.ds(sid * RPT, RPT)],
                        out_hbm.at[cid, pl.ds(sid * RPT, RPT)])

    return prop


def _make_count():
    @functools.partial(
        pl.kernel,
        out_type=jax.ShapeDtypeStruct((NC, NP, 16), jnp.float32),
        mesh=_mesh(),
        compiler_params=pltpu.CompilerParams(use_tc_tiling_on_sc=False),
        scratch_types=[
            pltpu.VMEM((BLK,), jnp.int32),
            pltpu.VMEM((BLK, 16), jnp.float32),
            pltpu.VMEM((RPT, 16), jnp.float32),
            pltpu.VMEM_SHARED((NP, 16), jnp.float32),
        ],
    )
    def count(dst_hbm, out_hbm, dst_v, ones_v, zb_v, acc_sh):
        cid = lax.axis_index("c")
        sid = lax.axis_index("s")
        wid = cid * NS + sid

        zvec = jnp.zeros((16,), jnp.float32)
        ovec = jnp.ones((16,), jnp.float32)

        def zrow(r, carry):
            zb_v[r, pl.ds(0, 16)] = zvec
            return carry

        lax.fori_loop(0, RPT, zrow, 0)

        def orow(r, carry):
            ones_v[r, pl.ds(0, 16)] = ovec
            return carry

        lax.fori_loop(0, BLK, orow, 0)

        pltpu.sync_copy(zb_v, acc_sh.at[pl.ds(sid * RPT, RPT)])
        plsc.subcore_barrier()

        ebase = wid * EPT

        def step(j, carry):
            pltpu.sync_copy(dst_hbm.at[pl.ds(ebase + j * BLK, BLK)], dst_v)
            pltpu.sync_copy(ones_v, acc_sh.at[dst_v], add=True)
            return carry

        lax.fori_loop(0, NB, step, 0)

        plsc.subcore_barrier()
        pltpu.sync_copy(acc_sh.at[pl.ds(sid * RPT, RPT)],
                        out_hbm.at[cid, pl.ds(sid * RPT, RPT)])

    return count


_count = _make_count()
_prop16 = _make_prop(H)
_prop64 = _make_prop(C)


def _tc1_body(cnt_ref, x_ref, w1_ref, g1_ref, dinv_ref):
    cnt = cnt_ref[0, 0:N, 0:1] + cnt_ref[1, 0:N, 0:1]
    deg = cnt + 1.0
    dinv = lax.rsqrt(jnp.maximum(deg, 1e-12))
    h0 = jnp.dot(x_ref[...], w1_ref[...], preferred_element_type=jnp.float32)
    g1_ref[...] = dinv * h0
    dinv_ref[...] = dinv


def _tc1(cnt, x, w1):
    return pl.pallas_call(
        _tc1_body,
        out_shape=(
            jax.ShapeDtypeStruct((N, H), jnp.float32),
            jax.ShapeDtypeStruct((N, 1), jnp.float32),
        ),
    )(cnt, x, w1)


def _tc_mid_body(s_ref, g_ref, dinv_ref, b_ref, w_ref, out_ref):
    dinv = dinv_ref[...]
    t = dinv * (s_ref[0, 0:N, :] + s_ref[1, 0:N, :] + g_ref[...]) + b_ref[...]
    m = jnp.sum(t, axis=0, keepdims=True) * (1.0 / N)
    t = t - m
    rn2 = jnp.sum(t * t) * (1.0 / N)
    t = t / jnp.sqrt(rn2 + 1e-6)
    t = jnp.maximum(t, 0.0)
    out_ref[...] = dinv * jnp.dot(t, w_ref[...],
                                  preferred_element_type=jnp.float32)


def _tc_mid(s, g, dinv, b, w):
    return pl.pallas_call(
        _tc_mid_body,
        out_shape=jax.ShapeDtypeStruct((N, w.shape[1]), jnp.float32),
    )(s, g, dinv, b, w)


def _tc_out_body(s_ref, g_ref, dinv_ref, b_ref, out_ref):
    t = dinv_ref[...] * (s_ref[0, 0:N, :] + s_ref[1, 0:N, :] + g_ref[...]) + b_ref[...]
    mx = jnp.max(t, axis=1, keepdims=True)
    t = t - mx
    lse = jnp.log(jnp.sum(jnp.exp(t), axis=1, keepdims=True))
    out_ref[...] = t - lse


def _tc_out(s, g, dinv, b):
    return pl.pallas_call(
        _tc_out_body,
        out_shape=jax.ShapeDtypeStruct((N, C), jnp.float32),
    )(s, g, dinv, b)


def kernel(x, edge_index, W1, b1, W2, b2, W3, b3):
    src = edge_index[0]
    dst = edge_index[1]
    cnt = _count(dst)
    g1, dinv = _tc1(cnt, x, W1)
    s1 = _prop16(g1, src, dst)
    g2 = _tc_mid(s1, g1, dinv, b1.reshape(1, H), W2)
    s2 = _prop16(g2, src, dst)
    g3 = _tc_mid(s2, g2, dinv, b2.reshape(1, H), W3)
    s3 = _prop64(g3, src, dst)
    return _tc_out(s3, g3, dinv, b3.reshape(1, C))


# speedup vs baseline: 33.2743x; 2.4720x over previous
import functools

import jax
import jax.numpy as jnp
from jax import lax
from jax.experimental import pallas as pl
from jax.experimental.pallas import tpu as pltpu
from jax.experimental.pallas import tpu_sc as plsc

N = 10000
E = 320000
D_IN = 128
H = 16
C = 64

NC = 2
NS = 16
NW = NC * NS
EPT = E // NW
BLK = 80
NB = EPT // BLK
NP = 10240
RPT = NP // NS

_mesh = lambda: plsc.VectorSubcoreMesh(core_axis_name="c", subcore_axis_name="s")


def _make_prop(W):
    @functools.partial(
        pl.kernel,
        out_type=jax.ShapeDtypeStruct((NC, NP, W), jnp.float32),
        mesh=_mesh(),
        compiler_params=pltpu.CompilerParams(use_tc_tiling_on_sc=False),
        scratch_types=[
            pltpu.VMEM((NB, BLK), jnp.int32),
            pltpu.VMEM((NB, BLK), jnp.int32),
            pltpu.VMEM((2, BLK, W), jnp.float32),
            pltpu.VMEM((RPT, W), jnp.float32),
            pltpu.VMEM_SHARED((NP, W), jnp.float32),
            pltpu.SemaphoreType.DMA((2,)),
        ],
    )
    def prop(g_hbm, src_hbm, dst_hbm, out_hbm, srcs_v, dsts_v, rows_v, zb_v,
             acc_sh, sem):
        cid = lax.axis_index("c")
        sid = lax.axis_index("s")
        wid = cid * NS + sid

        zvec = jnp.zeros((16,), jnp.float32)

        def zrow(r, carry):
            for c in range(W // 16):
                zb_v[r, pl.ds(c * 16, 16)] = zvec
            return carry

        lax.fori_loop(0, RPT, zrow, 0)
        pltpu.sync_copy(zb_v, acc_sh.at[pl.ds(sid * RPT, RPT)])
        pltpu.sync_copy(src_hbm.at[wid], srcs_v)
        pltpu.sync_copy(dst_hbm.at[wid], dsts_v)
        plsc.subcore_barrier()

        pltpu.async_copy(g_hbm.at[srcs_v.at[0]], rows_v.at[0], sem.at[0])
        pltpu.async_copy(g_hbm.at[srcs_v.at[1]], rows_v.at[1], sem.at[1])

        def step(j, carry):
            b = lax.rem(j, 2)
            pltpu.make_async_copy(g_hbm.at[srcs_v.at[j]], rows_v.at[b],
                                  sem.at[b]).wait()
            pltpu.sync_copy(rows_v.at[b], acc_sh.at[dsts_v.at[j]], add=True)

            @pl.when(j + 2 < NB)
            def _():
                pltpu.async_copy(g_hbm.at[srcs_v.at[j + 2]], rows_v.at[b],
                                 sem.at[b])

            return carry

        lax.fori_loop(0, NB, step, 0)

        plsc.subcore_barrier()
        pltpu.sync_copy(acc_sh.at[pl.ds(sid * RPT, RPT)],
                        out_hbm.at[cid, pl.ds(sid * RPT, RPT)])

    return prop


def _make_count():
    @functools.partial(
        pl.kernel,
        out_type=jax.ShapeDtypeStruct((NC, NP, 16), jnp.float32),
        mesh=_mesh(),
        compiler_params=pltpu.CompilerParams(use_tc_tiling_on_sc=False),
        scratch_types=[
            pltpu.VMEM((NB, BLK), jnp.int32),
            pltpu.VMEM((BLK, 16), jnp.float32),
            pltpu.VMEM((RPT, 16), jnp.float32),
            pltpu.VMEM_SHARED((NP, 16), jnp.float32),
        ],
    )
    def count(dst_hbm, out_hbm, dsts_v, ones_v, zb_v, acc_sh):
        cid = lax.axis_index("c")
        sid = lax.axis_index("s")
        wid = cid * NS + sid

        zvec = jnp.zeros((16,), jnp.float32)
        ovec = jnp.ones((16,), jnp.float32)

        def zrow(r, carry):
            zb_v[r, pl.ds(0, 16)] = zvec
            return carry

        lax.fori_loop(0, RPT, zrow, 0)

        def orow(r, carry):
            ones_v[r, pl.ds(0, 16)] = ovec
            return carry

        lax.fori_loop(0, BLK, orow, 0)

        pltpu.sync_copy(zb_v, acc_sh.at[pl.ds(sid * RPT, RPT)])
        pltpu.sync_copy(dst_hbm.at[wid], dsts_v)
        plsc.subcore_barrier()

        def step(j, carry):
            pltpu.sync_copy(ones_v, acc_sh.at[dsts_v.at[j]], add=True)
            return carry

        lax.fori_loop(0, NB, step, 0)

        plsc.subcore_barrier()
        pltpu.sync_copy(acc_sh.at[pl.ds(sid * RPT, RPT)],
                        out_hbm.at[cid, pl.ds(sid * RPT, RPT)])

    return count


_count = _make_count()
_prop16 = _make_prop(H)
_prop64 = _make_prop(C)


def _tc1_body(cnt_ref, x_ref, w1_ref, g1_ref, dinv_ref):
    cnt = cnt_ref[0, 0:N, 0:1] + cnt_ref[1, 0:N, 0:1]
    deg = cnt + 1.0
    dinv = lax.rsqrt(jnp.maximum(deg, 1e-12))
    h0 = jnp.dot(x_ref[...], w1_ref[...], preferred_element_type=jnp.float32)
    g1_ref[...] = dinv * h0
    dinv_ref[...] = dinv


def _tc1(cnt, x, w1):
    return pl.pallas_call(
        _tc1_body,
        out_shape=(
            jax.ShapeDtypeStruct((N, H), jnp.float32),
            jax.ShapeDtypeStruct((N, 1), jnp.float32),
        ),
    )(cnt, x, w1)


def _tc_mid_body(s_ref, g_ref, dinv_ref, b_ref, w_ref, out_ref):
    dinv = dinv_ref[...]
    t = dinv * (s_ref[0, 0:N, :] + s_ref[1, 0:N, :] + g_ref[...]) + b_ref[...]
    m = jnp.sum(t, axis=0, keepdims=True) * (1.0 / N)
    t = t - m
    rn2 = jnp.sum(t * t) * (1.0 / N)
    t = t / jnp.sqrt(rn2 + 1e-6)
    t = jnp.maximum(t, 0.0)
    out_ref[...] = dinv * jnp.dot(t, w_ref[...],
                                  preferred_element_type=jnp.float32)


def _tc_mid(s, g, dinv, b, w):
    return pl.pallas_call(
        _tc_mid_body,
        out_shape=jax.ShapeDtypeStruct((N, w.shape[1]), jnp.float32),
    )(s, g, dinv, b, w)


def _tc_out_body(s_ref, g_ref, dinv_ref, b_ref, out_ref):
    t = dinv_ref[...] * (s_ref[0, 0:N, :] + s_ref[1, 0:N, :] + g_ref[...]) + b_ref[...]
    mx = jnp.max(t, axis=1, keepdims=True)
    t = t - mx
    lse = jnp.log(jnp.sum(jnp.exp(t), axis=1, keepdims=True))
    out_ref[...] = t - lse


def _tc_out(s, g, dinv, b):
    return pl.pallas_call(
        _tc_out_body,
        out_shape=jax.ShapeDtypeStruct((N, C), jnp.float32),
    )(s, g, dinv, b)


def kernel(x, edge_index, W1, b1, W2, b2, W3, b3):
    src = edge_index[0].reshape(NW, NB, BLK)
    dst = edge_index[1].reshape(NW, NB, BLK)
    cnt = _count(dst)
    g1, dinv = _tc1(cnt, x, W1)
    s1 = _prop16(g1, src, dst)
    g2 = _tc_mid(s1, g1, dinv, b1.reshape(1, H), W2)
    s2 = _prop16(g2, src, dst)
    g3 = _tc_mid(s2, g2, dinv, b2.reshape(1, H), W3)
    s3 = _prop64(g3, src, dst)
    return _tc_out(s3, g3, dinv, b3.reshape(1, C))


# speedup vs baseline: 41.3088x; 1.2415x over previous
import functools

import jax
import jax.numpy as jnp
from jax import lax
from jax.experimental import pallas as pl
from jax.experimental.pallas import tpu as pltpu
from jax.experimental.pallas import tpu_sc as plsc

N = 10000
E = 320000
D_IN = 128
H = 16
C = 64

NC = 2
NS = 16
NW = NC * NS
EPT = E // NW
BLK = 80
NB = EPT // BLK
NP = 10240
RPT = NP // NS

_mesh = lambda: plsc.VectorSubcoreMesh(core_axis_name="c", subcore_axis_name="s")


def _make_prop(W):
    @functools.partial(
        pl.kernel,
        out_type=jax.ShapeDtypeStruct((NC, NP, W), jnp.float32),
        mesh=_mesh(),
        compiler_params=pltpu.CompilerParams(use_tc_tiling_on_sc=False),
        scratch_types=[
            pltpu.VMEM((NB, BLK), jnp.int32),
            pltpu.VMEM((NB, BLK), jnp.int32),
            pltpu.VMEM((4, BLK, W), jnp.float32),
            pltpu.VMEM((RPT, W), jnp.float32),
            pltpu.VMEM_SHARED((NP, W), jnp.float32),
            pltpu.SemaphoreType.DMA((4,)),
            pltpu.SemaphoreType.DMA((4,)),
        ],
    )
    def prop(g_hbm, src_hbm, dst_hbm, out_hbm, srcs_v, dsts_v, rows_v, zb_v,
             acc_sh, gsem, ssem):
        cid = lax.axis_index("c")
        sid = lax.axis_index("s")
        wid = cid * NS + sid

        zvec = jnp.zeros((16,), jnp.float32)

        def zrow(r, carry):
            for c in range(W // 16):
                zb_v[r, pl.ds(c * 16, 16)] = zvec
            return carry

        lax.fori_loop(0, RPT, zrow, 0)
        pltpu.sync_copy(zb_v, acc_sh.at[pl.ds(sid * RPT, RPT)])
        pltpu.sync_copy(src_hbm.at[wid], srcs_v)
        pltpu.sync_copy(dst_hbm.at[wid], dsts_v)
        plsc.subcore_barrier()

        pltpu.async_copy(g_hbm.at[srcs_v.at[0]], rows_v.at[0], gsem.at[0])
        pltpu.async_copy(g_hbm.at[srcs_v.at[1]], rows_v.at[1], gsem.at[1])

        def step(j, carry):
            b = lax.rem(j, 4)
            b2 = lax.rem(j + 2, 4)

            @pl.when(j >= 2)
            def _():
                pltpu.make_async_copy(rows_v.at[b2],
                                      acc_sh.at[dsts_v.at[j - 2]],
                                      ssem.at[b2]).wait()

            @pl.when(j + 2 < NB)
            def _():
                pltpu.async_copy(g_hbm.at[srcs_v.at[j + 2]], rows_v.at[b2],
                                 gsem.at[b2])

            pltpu.make_async_copy(g_hbm.at[srcs_v.at[j]], rows_v.at[b],
                                  gsem.at[b]).wait()
            pltpu.async_copy(rows_v.at[b], acc_sh.at[dsts_v.at[j]],
                             ssem.at[b], add=True)
            return carry

        lax.fori_loop(0, NB, step, 0)

        for jj in (NB - 2, NB - 1):
            pltpu.make_async_copy(rows_v.at[jj % 4],
                                  acc_sh.at[dsts_v.at[jj]],
                                  ssem.at[jj % 4]).wait()

        plsc.subcore_barrier()
        pltpu.sync_copy(acc_sh.at[pl.ds(sid * RPT, RPT)],
                        out_hbm.at[cid, pl.ds(sid * RPT, RPT)])

    return prop


def _make_count():
    @functools.partial(
        pl.kernel,
        out_type=jax.ShapeDtypeStruct((NC, NP, 16), jnp.float32),
        mesh=_mesh(),
        compiler_params=pltpu.CompilerParams(use_tc_tiling_on_sc=False),
        scratch_types=[
            pltpu.VMEM((NB, BLK), jnp.int32),
            pltpu.VMEM((BLK, 16), jnp.float32),
            pltpu.VMEM((RPT, 16), jnp.float32),
            pltpu.VMEM_SHARED((NP, 16), jnp.float32),
            pltpu.SemaphoreType.DMA,
        ],
    )
    def count(dst_hbm, out_hbm, dsts_v, ones_v, zb_v, acc_sh, ssem):
        cid = lax.axis_index("c")
        sid = lax.axis_index("s")
        wid = cid * NS + sid

        zvec = jnp.zeros((16,), jnp.float32)
        ovec = jnp.ones((16,), jnp.float32)

        def zrow(r, carry):
            zb_v[r, pl.ds(0, 16)] = zvec
            return carry

        lax.fori_loop(0, RPT, zrow, 0)

        def orow(r, carry):
            ones_v[r, pl.ds(0, 16)] = ovec
            return carry

        lax.fori_loop(0, BLK, orow, 0)

        pltpu.sync_copy(zb_v, acc_sh.at[pl.ds(sid * RPT, RPT)])
        pltpu.sync_copy(dst_hbm.at[wid], dsts_v)
        plsc.subcore_barrier()

        def step(j, carry):
            pltpu.async_copy(ones_v, acc_sh.at[dsts_v.at[j]], ssem, add=True)

            @pl.when(j >= 4)
            def _():
                pltpu.make_async_copy(ones_v, acc_sh.at[dsts_v.at[j]],
                                      ssem).wait()

            return carry

        lax.fori_loop(0, NB, step, 0)
        for _ in range(4):
            pltpu.make_async_copy(ones_v, acc_sh.at[dsts_v.at[0]], ssem).wait()

        plsc.subcore_barrier()
        pltpu.sync_copy(acc_sh.at[pl.ds(sid * RPT, RPT)],
                        out_hbm.at[cid, pl.ds(sid * RPT, RPT)])

    return count


_count = _make_count()
_prop16 = _make_prop(H)
_prop64 = _make_prop(C)


def _tc1_body(cnt_ref, x_ref, w1_ref, g1_ref, dinv_ref):
    cnt = cnt_ref[0, 0:N, 0:1] + cnt_ref[1, 0:N, 0:1]
    deg = cnt + 1.0
    dinv = lax.rsqrt(jnp.maximum(deg, 1e-12))
    h0 = jnp.dot(x_ref[...], w1_ref[...], preferred_element_type=jnp.float32)
    g1_ref[...] = dinv * h0
    dinv_ref[...] = dinv


def _tc1(cnt, x, w1):
    return pl.pallas_call(
        _tc1_body,
        out_shape=(
            jax.ShapeDtypeStruct((N, H), jnp.float32),
            jax.ShapeDtypeStruct((N, 1), jnp.float32),
        ),
    )(cnt, x, w1)


def _tc_mid_body(s_ref, g_ref, dinv_ref, b_ref, w_ref, out_ref):
    dinv = dinv_ref[...]
    t = dinv * (s_ref[0, 0:N, :] + s_ref[1, 0:N, :] + g_ref[...]) + b_ref[...]
    m = jnp.sum(t, axis=0, keepdims=True) * (1.0 / N)
    t = t - m
    rn2 = jnp.sum(t * t) * (1.0 / N)
    t = t / jnp.sqrt(rn2 + 1e-6)
    t = jnp.maximum(t, 0.0)
    out_ref[...] = dinv * jnp.dot(t, w_ref[...],
                                  preferred_element_type=jnp.float32)


def _tc_mid(s, g, dinv, b, w):
    return pl.pallas_call(
        _tc_mid_body,
        out_shape=jax.ShapeDtypeStruct((N, w.shape[1]), jnp.float32),
    )(s, g, dinv, b, w)


def _tc_out_body(s_ref, g_ref, dinv_ref, b_ref, out_ref):
    t = dinv_ref[...] * (s_ref[0, 0:N, :] + s_ref[1, 0:N, :] + g_ref[...]) + b_ref[...]
    mx = jnp.max(t, axis=1, keepdims=True)
    t = t - mx
    lse = jnp.log(jnp.sum(jnp.exp(t), axis=1, keepdims=True))
    out_ref[...] = t - lse


def _tc_out(s, g, dinv, b):
    return pl.pallas_call(
        _tc_out_body,
        out_shape=jax.ShapeDtypeStruct((N, C), jnp.float32),
    )(s, g, dinv, b)


def kernel(x, edge_index, W1, b1, W2, b2, W3, b3):
    src = edge_index[0].reshape(NW, NB, BLK)
    dst = edge_index[1].reshape(NW, NB, BLK)
    cnt = _count(dst)
    g1, dinv = _tc1(cnt, x, W1)
    s1 = _prop16(g1, src, dst)
    g2 = _tc_mid(s1, g1, dinv, b1.reshape(1, H), W2)
    s2 = _prop16(g2, src, dst)
    g3 = _tc_mid(s2, g2, dinv, b2.reshape(1, H), W3)
    s3 = _prop64(g3, src, dst)
    return _tc_out(s3, g3, dinv, b3.reshape(1, C))
